# baseline (speedup 1.0000x reference)
import functools

import jax
import jax.numpy as jnp
from jax import lax
from jax.experimental import pallas as pl
from jax.experimental.pallas import tpu as pltpu
from jax.experimental.pallas import tpu_sc as plsc

N_NODES = 10000
FEAT = 128
L = 16
NW = 32
ROWS = 320
PAD_ROW = ROWS
N_OUT_PAD = NW * ROWS
CHUNK = 12800
GROUPS = CHUNK // L
CUNROLL = 4
GB = 32
NBUF = 8
NCHUNKS = 320000 // CHUNK
LB = 1024
CPAD = ((CHUNK + GB + LB - 1) // LB) * LB
assert CPAD >= CHUNK + GB
FCOLS = FEAT // L


def _sc_mesh():
    return plsc.VectorSubcoreMesh(core_axis_name="c", subcore_axis_name="s")


def _emit_zero_acc(acc):
    zero = jnp.zeros((L,), jnp.float32)

    def zero_body(i, carry):
        for c in range(FCOLS):
            acc[i, pl.ds(c * L, L)] = zero
        return carry

    lax.fori_loop(0, ROWS + 1, zero_body, 0)


def _emit_gather_rmw(table_h, csrc, cdst, acc, rows, sems, nb):
    def fire(b):
        sel = b % NBUF
        pltpu.async_copy(table_h.at[csrc.at[pl.ds(b * GB, GB)]],
                         rows.at[sel], sems.at[sel])

    def wait(b):
        sel = b % NBUF
        pltpu.make_async_copy(table_h.at[csrc.at[pl.ds(b * GB, GB)]],
                              rows.at[sel], sems.at[sel]).wait()

    def process(b):
        sel = b % NBUF
        for jj in range(GB // L):
            dv = cdst[pl.ds(b * GB + jj * L, L)]
            for j16 in range(L):
                dj = dv[j16]
                j = jj * L + j16
                for c in range(FCOLS):
                    a = acc[dj, pl.ds(c * L, L)]
                    v = rows[sel, j, pl.ds(c * L, L)]
                    acc[dj, pl.ds(c * L, L)] = jnp.maximum(a, v)

    for i in range(NBUF - 1):
        @pl.when(i < nb)
        def _():
            fire(i)

    def blk(b, carry2):
        @pl.when(b + NBUF - 1 < nb)
        def _():
            fire(b + NBUF - 1)

        wait(b)
        process(b)
        return carry2

    lax.fori_loop(0, nb, blk, 0)


_SC_SCRATCH = [
    pltpu.VMEM((ROWS + 1, FEAT), jnp.float32),
    pltpu.VMEM((CPAD,), jnp.int32),
    pltpu.VMEM((CPAD,), jnp.int32),
    pltpu.VMEM((NBUF, GB, FEAT), jnp.float32),
    pltpu.VMEM((NCHUNKS * L,), jnp.int32),
    pltpu.SemaphoreType.DMA((NBUF,)),
    pltpu.SemaphoreType.DMA,
]


def _sc_scatter_max_scan(table, src, dst):
    E = src.shape[0]
    n_chunks = E // CHUNK
    assert n_chunks == NCHUNKS

    def body(table_h, src_h, dst_h, out_h, csp_h, dsp_h, ne_h,
             acc, csrc, cdst, rows, nea, sems, sem_sv, srcb, dstb):
        wid = lax.axis_index("s") * 2 + lax.axis_index("c")
        lo = wid * ROWS
        iota = lax.broadcasted_iota(jnp.int32, (L,), 0)
        _emit_zero_acc(acc)

        def chunk_body(k, carry):
            pltpu.sync_copy(src_h.at[pl.ds(k * CHUNK, CHUNK)], srcb)
            pltpu.sync_copy(dst_h.at[pl.ds(k * CHUNK, CHUNK)], dstb)

            def grp(g, ptr):
                base = ptr
                tot = None
                for t in range(CUNROLL):
                    d = dstb[pl.ds((g * CUNROLL + t) * L, L)]
                    s = srcb[pl.ds((g * CUNROLL + t) * L, L)]
                    m = (d >= lo) & (d < lo + ROWS)
                    pos = plsc.cumsum(m.astype(jnp.int32))
                    offs = base + pos - 1
                    plsc.store_scatter(csrc, [offs], s, mask=m)
                    plsc.store_scatter(cdst, [offs], d - lo, mask=m)
                    cnt = plsc.all_reduce_population_count(m)
                    base = base + cnt
                    tot = cnt if tot is None else tot + cnt
                return ptr + jnp.max(tot)

            ne = lax.fori_loop(0, GROUPS // CUNROLL, grp, jnp.int32(0))

            pad_src = lo + iota
            pad_dst = jnp.full((L,), PAD_ROW, jnp.int32)
            for t in range(GB // L):
                csrc[pl.ds(ne + t * L, L)] = pad_src
                cdst[pl.ds(ne + t * L, L)] = pad_dst
            nb = (ne + GB - 1) // GB

            nea[pl.ds(k * L, L)] = jnp.full((L,), ne, jnp.int32)
            nlb = (nb * GB + LB - 1) // LB

            def save(i, c2):
                pltpu.async_copy(csrc.at[pl.ds(i * LB, LB)],
                                 csp_h.at[wid, pl.ds(k * CPAD + i * LB, LB)],
                                 sem_sv)
                pltpu.async_copy(cdst.at[pl.ds(i * LB, LB)],
                                 dsp_h.at[wid, pl.ds(k * CPAD + i * LB, LB)],
                                 sem_sv)
                return c2

            lax.fori_loop(0, nlb, save, 0)

            _emit_gather_rmw(table_h, csrc, cdst, acc, rows, sems, nb)

            def drain(i, c2):
                pltpu.make_async_copy(
                    csrc.at[pl.ds(i * LB, LB)],
                    csp_h.at[wid, pl.ds(k * CPAD + i * LB, LB)], sem_sv).wait()
                pltpu.make_async_copy(
                    cdst.at[pl.ds(i * LB, LB)],
                    dsp_h.at[wid, pl.ds(k * CPAD + i * LB, LB)], sem_sv).wait()
                return c2

            lax.fori_loop(0, nlb, drain, 0)
            return carry

        lax.fori_loop(0, n_chunks, chunk_body, 0)
        pltpu.sync_copy(nea, ne_h.at[wid])
        pltpu.sync_copy(acc.at[pl.ds(0, ROWS)], out_h.at[pl.ds(lo, ROWS)])

    return pl.kernel(
        body,
        out_type=[
            jax.ShapeDtypeStruct((N_OUT_PAD, FEAT), jnp.float32),
            jax.ShapeDtypeStruct((NW, NCHUNKS * CPAD), jnp.int32),
            jax.ShapeDtypeStruct((NW, NCHUNKS * CPAD), jnp.int32),
            jax.ShapeDtypeStruct((NW, NCHUNKS * L), jnp.int32),
        ],
        mesh=_sc_mesh(),
        compiler_params=pltpu.CompilerParams(needs_layout_passes=False),
        scratch_types=_SC_SCRATCH + [
            pltpu.VMEM((CHUNK,), jnp.int32),
            pltpu.VMEM((CHUNK,), jnp.int32),
        ],
    )(table, src, dst)


def _sc_scatter_max_replay(table, csp, dsp, nesp):
    def body(table_h, csp_h, dsp_h, ne_h, out_h,
             acc, csrc, cdst, rows, nea, sems, sem_sv):
        wid = lax.axis_index("s") * 2 + lax.axis_index("c")
        lo = wid * ROWS
        pltpu.sync_copy(ne_h.at[wid], nea)
        _emit_zero_acc(acc)

        def chunk_body(k, carry):
            nv = nea[pl.ds(k * L, L)]
            ne = nv[0]
            nb = (ne + GB - 1) // GB
            nlb = (nb * GB + LB - 1) // LB

            def load(i, c2):
                pltpu.sync_copy(csp_h.at[wid, pl.ds(k * CPAD + i * LB, LB)],
                                csrc.at[pl.ds(i * LB, LB)])
                pltpu.sync_copy(dsp_h.at[wid, pl.ds(k * CPAD + i * LB, LB)],
                                cdst.at[pl.ds(i * LB, LB)])
                return c2

            lax.fori_loop(0, nlb, load, 0)

            _emit_gather_rmw(table_h, csrc, cdst, acc, rows, sems, nb)
            return carry

        lax.fori_loop(0, NCHUNKS, chunk_body, 0)
        pltpu.sync_copy(acc.at[pl.ds(0, ROWS)], out_h.at[pl.ds(lo, ROWS)])

    return pl.kernel(
        body,
        out_type=jax.ShapeDtypeStruct((N_OUT_PAD, FEAT), jnp.float32),
        mesh=_sc_mesh(),
        compiler_params=pltpu.CompilerParams(needs_layout_passes=False),
        scratch_types=_SC_SCRATCH,
    )(table, csp, dsp, nesp)


def _sc_gather(table, idx):
    B = idx.shape[0]
    bpw = B // NW

    def body(t_h, i_h, o_h, idxv, rowsv, sem):
        wid = lax.axis_index("s") * 2 + lax.axis_index("c")
        base = wid * bpw
        pltpu.sync_copy(i_h.at[pl.ds(base, bpw)], idxv)
        pltpu.async_copy(t_h.at[idxv], rowsv, sem).wait()
        pltpu.sync_copy(rowsv, o_h.at[pl.ds(base, bpw)])

    return pl.kernel(
        body,
        out_type=jax.ShapeDtypeStruct((B, FEAT), jnp.float32),
        mesh=_sc_mesh(),
        compiler_params=pltpu.CompilerParams(needs_layout_passes=False),
        scratch_types=[
            pltpu.VMEM((bpw,), jnp.int32),
            pltpu.VMEM((bpw, FEAT), jnp.float32),
            pltpu.SemaphoreType.DMA,
        ],
    )(table, idx)


_BM = 1000


def _full_spec(shape):
    return pl.BlockSpec(shape, lambda i: (0,) * len(shape))


def _row_spec(cols):
    return pl.BlockSpec((_BM, cols), lambda i: (i, 0))


def _pool_body(x_ref, w_ref, b_ref, o_ref):
    o_ref[...] = jnp.maximum(
        jnp.dot(x_ref[...], w_ref[...], preferred_element_type=jnp.float32)
        + b_ref[...], 0.0)


def _pool_mm(x, w, b):
    n = x.shape[0]
    return pl.pallas_call(
        _pool_body,
        grid=(n // _BM,),
        in_specs=[_row_spec(x.shape[1]), _full_spec(w.shape),
                  _full_spec((1, w.shape[1]))],
        out_specs=_row_spec(w.shape[1]),
        out_shape=jax.ShapeDtypeStruct((n, w.shape[1]), jnp.float32),
    )(x, w, b.reshape(1, -1))


def _layer1_body(x_ref, n_ref, ws_ref, wn_ref, b_ref, wp_ref, bp_ref,
                 h_ref, f_ref):
    h = (jnp.dot(x_ref[...], ws_ref[...], preferred_element_type=jnp.float32)
         + jnp.dot(n_ref[...], wn_ref[...], preferred_element_type=jnp.float32)
         + b_ref[...])
    h = jax.nn.sigmoid(h)
    h_ref[...] = h
    f_ref[...] = jnp.maximum(
        jnp.dot(h, wp_ref[...], preferred_element_type=jnp.float32)
        + bp_ref[...], 0.0)


def _layer1_fused(x, neigh, w_self, w_neigh, bias, w_pool2, b_pool2):
    n = x.shape[0]
    return pl.pallas_call(
        _layer1_body,
        grid=(n // _BM,),
        in_specs=[_row_spec(FEAT), _row_spec(FEAT), _full_spec((FEAT, FEAT)),
                  _full_spec((FEAT, FEAT)), _full_spec((1, FEAT)),
                  _full_spec((FEAT, FEAT)), _full_spec((1, FEAT))],
        out_specs=[_row_spec(FEAT), _row_spec(FEAT)],
        out_shape=[jax.ShapeDtypeStruct((n, FEAT), jnp.float32),
                   jax.ShapeDtypeStruct((n, FEAT), jnp.float32)],
    )(x, neigh, w_self, w_neigh, bias.reshape(1, -1), w_pool2,
      b_pool2.reshape(1, -1))


def _layer2_body(x_ref, n_ref, ws_ref, wn_ref, b_ref, o_ref):
    o_ref[...] = (
        jnp.dot(x_ref[...], ws_ref[...], preferred_element_type=jnp.float32)
        + jnp.dot(n_ref[...], wn_ref[...], preferred_element_type=jnp.float32)
        + b_ref[...])


def _layer2(x, neigh, w_self, w_neigh, bias):
    n = x.shape[0]
    return pl.pallas_call(
        _layer2_body,
        grid=(n // _BM,),
        in_specs=[_row_spec(FEAT), _row_spec(FEAT), _full_spec((FEAT, FEAT)),
                  _full_spec((FEAT, FEAT)), _full_spec((1, FEAT))],
        out_specs=_row_spec(FEAT),
        out_shape=jax.ShapeDtypeStruct((n, FEAT), jnp.float32),
    )(x, neigh, w_self, w_neigh, bias.reshape(1, -1))


def _head_body(g_ref, c_ref, wc_ref, bc_ref, w1_ref, b1_ref, w2_ref, b2_ref,
               w3_ref, b3_ref, w4_ref, b4_ref, o_ref):
    cmd = c_ref[...]
    enc = (cmd[:, 0:1] * wc_ref[0:1, :]
           + cmd[:, 1:2] * wc_ref[1:2, :] + bc_ref[...])
    prod = g_ref[...] * enc
    o = jax.nn.sigmoid(
        jnp.dot(prod, w1_ref[...], preferred_element_type=jnp.float32)
        + b1_ref[...])
    o = jax.nn.sigmoid(
        jnp.dot(o, w2_ref[...], preferred_element_type=jnp.float32)
        + b2_ref[...])
    o = jax.nn.sigmoid(
        jnp.dot(o, w3_ref[...], preferred_element_type=jnp.float32)
        + b3_ref[...])
    o_ref[...] = jax.nn.sigmoid(
        jnp.dot(o, w4_ref[...], preferred_element_type=jnp.float32)
        + b4_ref[...])


def _head(g, command, w_cmd, b_cmd, w1, b1, w2, b2, w3, b3, w4, b4):
    n = g.shape[0]
    return pl.pallas_call(
        _head_body,
        out_shape=jax.ShapeDtypeStruct((n, 1), jnp.float32),
    )(g, command.reshape(1, 2), w_cmd, b_cmd.reshape(1, -1),
      w1, b1.reshape(1, -1), w2, b2.reshape(1, -1),
      w3, b3.reshape(1, -1), w4, b4.reshape(1, -1))


def kernel(node_inputs, edge_index, leaf_nodes, command,
           W_pool1, b_pool1, W_self1, W_neigh1, bias1,
           W_pool2, b_pool2, W_self2, W_neigh2, bias2,
           W_cmd, b_cmd, W_o1, b_o1, W_o2, b_o2, W_o3, b_o3, W_o4, b_o4):
    src = edge_index[0]
    dst = edge_index[1]

    f1 = _pool_mm(node_inputs, W_pool1, b_pool1)
    n1p, csp, dsp, nesp = _sc_scatter_max_scan(f1, src, dst)
    n1 = n1p[:N_NODES]
    h, f2 = _layer1_fused(node_inputs, n1, W_self1, W_neigh1, bias1,
                          W_pool2, b_pool2)
    n2 = _sc_scatter_max_replay(f2, csp, dsp, nesp)[:N_NODES]
    h2 = _layer2(h, n2, W_self2, W_neigh2, bias2)
    g = _sc_gather(h2, leaf_nodes)
    return _head(g, command, W_cmd, b_cmd, W_o1, b_o1, W_o2, b_o2,
                 W_o3, b_o3, W_o4, b_o4)


# speedup vs baseline: 1.0174x; 1.0174x over previous
import functools

import jax
import jax.numpy as jnp
from jax import lax
from jax.experimental import pallas as pl
from jax.experimental.pallas import tpu as pltpu
from jax.experimental.pallas import tpu_sc as plsc

N_NODES = 10000
FEAT = 128
L = 16
NW = 32
ROWS = 320
PAD_ROW = ROWS
N_OUT_PAD = NW * ROWS
CHUNK = 12800
GROUPS = CHUNK // L
CUNROLL = 4
GB = 32
NBUF = 4
NCHUNKS = 320000 // CHUNK
LB = 1024
CPAD = ((CHUNK + GB + LB - 1) // LB) * LB
assert CPAD >= CHUNK + GB
FCOLS = FEAT // L


def _sc_mesh():
    return plsc.VectorSubcoreMesh(core_axis_name="c", subcore_axis_name="s")


def _emit_zero_acc(acc):
    zero = jnp.zeros((L,), jnp.float32)

    def zero_body(i, carry):
        for c in range(FCOLS):
            acc[i, pl.ds(c * L, L)] = zero
        return carry

    lax.fori_loop(0, ROWS + 1, zero_body, 0)


def _emit_gather_rmw(table_h, csrc, cdst, acc, rows, sems, nb):
    def fire(b):
        sel = b % NBUF
        pltpu.async_copy(table_h.at[csrc.at[pl.ds(b * GB, GB)]],
                         rows.at[sel], sems.at[sel])

    def wait(b):
        sel = b % NBUF
        pltpu.make_async_copy(table_h.at[csrc.at[pl.ds(b * GB, GB)]],
                              rows.at[sel], sems.at[sel]).wait()

    def process(b):
        sel = b % NBUF
        for jj in range(GB // L):
            dv = cdst[pl.ds(b * GB + jj * L, L)]
            for j16 in range(L):
                dj = dv[j16]
                j = jj * L + j16
                for c in range(FCOLS):
                    a = acc[dj, pl.ds(c * L, L)]
                    v = rows[sel, j, pl.ds(c * L, L)]
                    acc[dj, pl.ds(c * L, L)] = jnp.maximum(a, v)

    for i in range(NBUF - 1):
        @pl.when(i < nb)
        def _():
            fire(i)

    def blk(b, carry2):
        @pl.when(b + NBUF - 1 < nb)
        def _():
            fire(b + NBUF - 1)

        wait(b)
        process(b)
        return carry2

    lax.fori_loop(0, nb, blk, 0)


_SC_SCRATCH = [
    pltpu.VMEM((ROWS + 1, FEAT), jnp.float32),
    pltpu.VMEM((CPAD,), jnp.int32),
    pltpu.VMEM((CPAD,), jnp.int32),
    pltpu.VMEM((NBUF, GB, FEAT), jnp.float32),
    pltpu.VMEM((NCHUNKS * L,), jnp.int32),
    pltpu.SemaphoreType.DMA((NBUF,)),
    pltpu.SemaphoreType.DMA,
]


def _sc_scatter_max_scan(table, src, dst):
    E = src.shape[0]
    n_chunks = E // CHUNK
    assert n_chunks == NCHUNKS

    def body(table_h, src_h, dst_h, out_h, csp_h, dsp_h, ne_h,
             acc, csrc, cdst, rows, nea, sems, sem_sv, srcb, dstb):
        wid = lax.axis_index("s") * 2 + lax.axis_index("c")
        lo = wid * ROWS
        iota = lax.broadcasted_iota(jnp.int32, (L,), 0)
        _emit_zero_acc(acc)

        def chunk_body(k, carry):
            pltpu.sync_copy(src_h.at[pl.ds(k * CHUNK, CHUNK)], srcb)
            pltpu.sync_copy(dst_h.at[pl.ds(k * CHUNK, CHUNK)], dstb)

            def grp(g, ptr):
                base = ptr
                tot = None
                for t in range(CUNROLL):
                    d = dstb[pl.ds((g * CUNROLL + t) * L, L)]
                    s = srcb[pl.ds((g * CUNROLL + t) * L, L)]
                    m = (d >= lo) & (d < lo + ROWS)
                    pos = plsc.cumsum(m.astype(jnp.int32))
                    offs = base + pos - 1
                    plsc.store_scatter(csrc, [offs], s, mask=m)
                    plsc.store_scatter(cdst, [offs], d - lo, mask=m)
                    cnt = plsc.all_reduce_population_count(m)
                    base = base + cnt
                    tot = cnt if tot is None else tot + cnt
                return ptr + jnp.max(tot)

            ne = lax.fori_loop(0, GROUPS // CUNROLL, grp, jnp.int32(0))

            pad_src = lo + iota
            pad_dst = jnp.full((L,), PAD_ROW, jnp.int32)
            for t in range(GB // L):
                csrc[pl.ds(ne + t * L, L)] = pad_src
                cdst[pl.ds(ne + t * L, L)] = pad_dst
            nb = (ne + GB - 1) // GB

            nea[pl.ds(k * L, L)] = jnp.full((L,), ne, jnp.int32)
            nlb = (nb * GB + LB - 1) // LB

            def save(i, c2):
                pltpu.async_copy(csrc.at[pl.ds(i * LB, LB)],
                                 csp_h.at[wid, pl.ds(k * CPAD + i * LB, LB)],
                                 sem_sv)
                pltpu.async_copy(cdst.at[pl.ds(i * LB, LB)],
                                 dsp_h.at[wid, pl.ds(k * CPAD + i * LB, LB)],
                                 sem_sv)
                return c2

            lax.fori_loop(0, nlb, save, 0)

            _emit_gather_rmw(table_h, csrc, cdst, acc, rows, sems, nb)

            def drain(i, c2):
                pltpu.make_async_copy(
                    csrc.at[pl.ds(i * LB, LB)],
                    csp_h.at[wid, pl.ds(k * CPAD + i * LB, LB)], sem_sv).wait()
                pltpu.make_async_copy(
                    cdst.at[pl.ds(i * LB, LB)],
                    dsp_h.at[wid, pl.ds(k * CPAD + i * LB, LB)], sem_sv).wait()
                return c2

            lax.fori_loop(0, nlb, drain, 0)
            return carry

        lax.fori_loop(0, n_chunks, chunk_body, 0)
        pltpu.sync_copy(nea, ne_h.at[wid])
        pltpu.sync_copy(acc.at[pl.ds(0, ROWS)], out_h.at[pl.ds(lo, ROWS)])

    return pl.kernel(
        body,
        out_type=[
            jax.ShapeDtypeStruct((N_OUT_PAD, FEAT), jnp.float32),
            jax.ShapeDtypeStruct((NW, NCHUNKS * CPAD), jnp.int32),
            jax.ShapeDtypeStruct((NW, NCHUNKS * CPAD), jnp.int32),
            jax.ShapeDtypeStruct((NW, NCHUNKS * L), jnp.int32),
        ],
        mesh=_sc_mesh(),
        compiler_params=pltpu.CompilerParams(needs_layout_passes=False),
        scratch_types=_SC_SCRATCH + [
            pltpu.VMEM((CHUNK,), jnp.int32),
            pltpu.VMEM((CHUNK,), jnp.int32),
        ],
    )(table, src, dst)


def _sc_scatter_max_replay(table, csp, dsp, nesp):
    def body(table_h, csp_h, dsp_h, ne_h, out_h,
             acc, csrc, cdst, rows, nea, sems, sem_sv):
        wid = lax.axis_index("s") * 2 + lax.axis_index("c")
        lo = wid * ROWS
        pltpu.sync_copy(ne_h.at[wid], nea)
        _emit_zero_acc(acc)

        def chunk_body(k, carry):
            nv = nea[pl.ds(k * L, L)]
            ne = nv[0]
            nb = (ne + GB - 1) // GB
            nlb = (nb * GB + LB - 1) // LB

            def load(i, c2):
                pltpu.sync_copy(csp_h.at[wid, pl.ds(k * CPAD + i * LB, LB)],
                                csrc.at[pl.ds(i * LB, LB)])
                pltpu.sync_copy(dsp_h.at[wid, pl.ds(k * CPAD + i * LB, LB)],
                                cdst.at[pl.ds(i * LB, LB)])
                return c2

            lax.fori_loop(0, nlb, load, 0)

            _emit_gather_rmw(table_h, csrc, cdst, acc, rows, sems, nb)
            return carry

        lax.fori_loop(0, NCHUNKS, chunk_body, 0)
        pltpu.sync_copy(acc.at[pl.ds(0, ROWS)], out_h.at[pl.ds(lo, ROWS)])

    return pl.kernel(
        body,
        out_type=jax.ShapeDtypeStruct((N_OUT_PAD, FEAT), jnp.float32),
        mesh=_sc_mesh(),
        compiler_params=pltpu.CompilerParams(needs_layout_passes=False),
        scratch_types=_SC_SCRATCH,
    )(table, csp, dsp, nesp)


def _sc_gather(table, idx):
    B = idx.shape[0]
    bpw = B // NW

    def body(t_h, i_h, o_h, idxv, rowsv, sem):
        wid = lax.axis_index("s") * 2 + lax.axis_index("c")
        base = wid * bpw
        pltpu.sync_copy(i_h.at[pl.ds(base, bpw)], idxv)
        pltpu.async_copy(t_h.at[idxv], rowsv, sem).wait()
        pltpu.sync_copy(rowsv, o_h.at[pl.ds(base, bpw)])

    return pl.kernel(
        body,
        out_type=jax.ShapeDtypeStruct((B, FEAT), jnp.float32),
        mesh=_sc_mesh(),
        compiler_params=pltpu.CompilerParams(needs_layout_passes=False),
        scratch_types=[
            pltpu.VMEM((bpw,), jnp.int32),
            pltpu.VMEM((bpw, FEAT), jnp.float32),
            pltpu.SemaphoreType.DMA,
        ],
    )(table, idx)


_BM = 1000


def _full_spec(shape):
    return pl.BlockSpec(shape, lambda i: (0,) * len(shape))


def _row_spec(cols):
    return pl.BlockSpec((_BM, cols), lambda i: (i, 0))


def _pool_body(x_ref, w_ref, b_ref, o_ref):
    o_ref[...] = jnp.maximum(
        jnp.dot(x_ref[...], w_ref[...], preferred_element_type=jnp.float32)
        + b_ref[...], 0.0)


def _pool_mm(x, w, b):
    n = x.shape[0]
    return pl.pallas_call(
        _pool_body,
        grid=(n // _BM,),
        in_specs=[_row_spec(x.shape[1]), _full_spec(w.shape),
                  _full_spec((1, w.shape[1]))],
        out_specs=_row_spec(w.shape[1]),
        out_shape=jax.ShapeDtypeStruct((n, w.shape[1]), jnp.float32),
    )(x, w, b.reshape(1, -1))


def _layer1_body(x_ref, n_ref, ws_ref, wn_ref, b_ref, wp_ref, bp_ref,
                 h_ref, f_ref):
    h = (jnp.dot(x_ref[...], ws_ref[...], preferred_element_type=jnp.float32)
         + jnp.dot(n_ref[...], wn_ref[...], preferred_element_type=jnp.float32)
         + b_ref[...])
    h = jax.nn.sigmoid(h)
    h_ref[...] = h
    f_ref[...] = jnp.maximum(
        jnp.dot(h, wp_ref[...], preferred_element_type=jnp.float32)
        + bp_ref[...], 0.0)


def _layer1_fused(x, neigh, w_self, w_neigh, bias, w_pool2, b_pool2):
    n = x.shape[0]
    return pl.pallas_call(
        _layer1_body,
        grid=(n // _BM,),
        in_specs=[_row_spec(FEAT), _row_spec(FEAT), _full_spec((FEAT, FEAT)),
                  _full_spec((FEAT, FEAT)), _full_spec((1, FEAT)),
                  _full_spec((FEAT, FEAT)), _full_spec((1, FEAT))],
        out_specs=[_row_spec(FEAT), _row_spec(FEAT)],
        out_shape=[jax.ShapeDtypeStruct((n, FEAT), jnp.float32),
                   jax.ShapeDtypeStruct((n, FEAT), jnp.float32)],
    )(x, neigh, w_self, w_neigh, bias.reshape(1, -1), w_pool2,
      b_pool2.reshape(1, -1))


def _layer2_body(x_ref, n_ref, ws_ref, wn_ref, b_ref, o_ref):
    o_ref[...] = (
        jnp.dot(x_ref[...], ws_ref[...], preferred_element_type=jnp.float32)
        + jnp.dot(n_ref[...], wn_ref[...], preferred_element_type=jnp.float32)
        + b_ref[...])


def _layer2(x, neigh, w_self, w_neigh, bias):
    n = x.shape[0]
    return pl.pallas_call(
        _layer2_body,
        grid=(n // _BM,),
        in_specs=[_row_spec(FEAT), _row_spec(FEAT), _full_spec((FEAT, FEAT)),
                  _full_spec((FEAT, FEAT)), _full_spec((1, FEAT))],
        out_specs=_row_spec(FEAT),
        out_shape=jax.ShapeDtypeStruct((n, FEAT), jnp.float32),
    )(x, neigh, w_self, w_neigh, bias.reshape(1, -1))


def _head_body(g_ref, c_ref, wc_ref, bc_ref, w1_ref, b1_ref, w2_ref, b2_ref,
               w3_ref, b3_ref, w4_ref, b4_ref, o_ref):
    cmd = c_ref[...]
    enc = (cmd[:, 0:1] * wc_ref[0:1, :]
           + cmd[:, 1:2] * wc_ref[1:2, :] + bc_ref[...])
    prod = g_ref[...] * enc
    o = jax.nn.sigmoid(
        jnp.dot(prod, w1_ref[...], preferred_element_type=jnp.float32)
        + b1_ref[...])
    o = jax.nn.sigmoid(
        jnp.dot(o, w2_ref[...], preferred_element_type=jnp.float32)
        + b2_ref[...])
    o = jax.nn.sigmoid(
        jnp.dot(o, w3_ref[...], preferred_element_type=jnp.float32)
        + b3_ref[...])
    o_ref[...] = jax.nn.sigmoid(
        jnp.dot(o, w4_ref[...], preferred_element_type=jnp.float32)
        + b4_ref[...])


def _head(g, command, w_cmd, b_cmd, w1, b1, w2, b2, w3, b3, w4, b4):
    n = g.shape[0]
    return pl.pallas_call(
        _head_body,
        out_shape=jax.ShapeDtypeStruct((n, 1), jnp.float32),
    )(g, command.reshape(1, 2), w_cmd, b_cmd.reshape(1, -1),
      w1, b1.reshape(1, -1), w2, b2.reshape(1, -1),
      w3, b3.reshape(1, -1), w4, b4.reshape(1, -1))


def kernel(node_inputs, edge_index, leaf_nodes, command,
           W_pool1, b_pool1, W_self1, W_neigh1, bias1,
           W_pool2, b_pool2, W_self2, W_neigh2, bias2,
           W_cmd, b_cmd, W_o1, b_o1, W_o2, b_o2, W_o3, b_o3, W_o4, b_o4):
    src = edge_index[0]
    dst = edge_index[1]

    f1 = _pool_mm(node_inputs, W_pool1, b_pool1)
    n1p, csp, dsp, nesp = _sc_scatter_max_scan(f1, src, dst)
    n1 = n1p[:N_NODES]
    h, f2 = _layer1_fused(node_inputs, n1, W_self1, W_neigh1, bias1,
                          W_pool2, b_pool2)
    n2 = _sc_scatter_max_replay(f2, csp, dsp, nesp)[:N_NODES]
    h2 = _layer2(h, n2, W_self2, W_neigh2, bias2)
    g = _sc_gather(h2, leaf_nodes)
    return _head(g, command, W_cmd, b_cmd, W_o1, b_o1, W_o2, b_o2,
                 W_o3, b_o3, W_o4, b_o4)
